# Initial kernel scaffold; baseline (speedup 1.0000x reference)
import jax
import jax.numpy as jnp
from jax.experimental import pallas as pl


def kernel(features, edge_index, W1, b1, W2, b2):
    raise NotImplementedError("write your pallas kernel here")


# speedup vs baseline: 5.4878x; 5.4878x over previous
import functools

import jax
import jax.numpy as jnp
from jax import lax
from jax.experimental import pallas as pl
from jax.experimental.pallas import tpu as pltpu
from jax.experimental.pallas import tpu_sc as plsc

N = 10000
E = 320000
DI = 128
DH = 256
HALF = DH // 2

NC = 2
NS = 16
EPT = E // NS
CH = 200
NCHUNK = EPT // CH
ST = 624
TAIL = N - NS * ST

_MESH = plsc.VectorSubcoreMesh(
    core_axis_name="c", subcore_axis_name="s", num_cores=NC, num_subcores=NS
)


@functools.partial(
    pl.kernel,
    out_type=jax.ShapeDtypeStruct((2 * N, 16), jnp.float32),
    mesh=_MESH,
    scratch_types=[
        pltpu.VMEM((CH,), jnp.int32),
        pltpu.VMEM((CH, 16), jnp.float32),
        pltpu.VMEM_SHARED((N, 16), jnp.float32),
    ],
    compiler_params=pltpu.CompilerParams(use_tc_tiling_on_sc=False),
)
def _deg_kernel(edges, zdeg, out, idxv, onesv, degsh):
    c = lax.axis_index("c")
    s = lax.axis_index("s")

    @pl.loop(0, CH)
    def _(i):
        onesv[i] = jnp.full((16,), 1.0, jnp.float32)

    pltpu.sync_copy(zdeg.at[pl.ds(s * ST, ST)], degsh.at[pl.ds(s * ST, ST)])

    @pl.when(s == NS - 1)
    def _():
        pltpu.sync_copy(
            zdeg.at[pl.ds(NS * ST, TAIL)], degsh.at[pl.ds(NS * ST, TAIL)]
        )

    plsc.subcore_barrier()

    base = c * (2 * E) + s * EPT

    @pl.loop(0, NCHUNK)
    def _(i):
        pltpu.sync_copy(edges.at[pl.ds(base + i * CH, CH)], idxv)
        pltpu.sync_copy(onesv, degsh.at[idxv], add=True)

    plsc.subcore_barrier()
    pltpu.sync_copy(
        degsh.at[pl.ds(s * ST, ST)], out.at[pl.ds(c * N + s * ST, ST)]
    )

    @pl.when(s == NS - 1)
    def _():
        pltpu.sync_copy(
            degsh.at[pl.ds(NS * ST, TAIL)], out.at[pl.ds(c * N + NS * ST, TAIL)]
        )


@functools.partial(
    pl.kernel,
    out_type=jax.ShapeDtypeStruct((2 * N, HALF), jnp.float32),
    mesh=_MESH,
    scratch_types=[
        pltpu.VMEM((CH,), jnp.int32),
        pltpu.VMEM((CH,), jnp.int32),
        pltpu.VMEM((CH, HALF), jnp.float32),
        pltpu.VMEM_SHARED((N, HALF), jnp.float32),
    ],
)
def _agg_kernel(h, edges, zagg, out, isrc, idst, rows, aggsh):
    c = lax.axis_index("c")
    s = lax.axis_index("s")

    pltpu.sync_copy(zagg.at[pl.ds(s * ST, ST)], aggsh.at[pl.ds(s * ST, ST)])

    @pl.when(s == NS - 1)
    def _():
        pltpu.sync_copy(
            zagg.at[pl.ds(NS * ST, TAIL)], aggsh.at[pl.ds(NS * ST, TAIL)]
        )

    plsc.subcore_barrier()

    base = s * EPT

    @pl.loop(0, NCHUNK)
    def _(i):
        pltpu.sync_copy(edges.at[pl.ds(c * E + base + i * CH, CH)], isrc)
        pltpu.sync_copy(edges.at[pl.ds(2 * E + base + i * CH, CH)], idst)
        pltpu.sync_copy(h.at[isrc], rows)
        pltpu.sync_copy(rows, aggsh.at[idst], add=True)

    plsc.subcore_barrier()
    pltpu.sync_copy(
        aggsh.at[pl.ds(s * ST, ST)], out.at[pl.ds(c * N + s * ST, ST)]
    )

    @pl.when(s == NS - 1)
    def _():
        pltpu.sync_copy(
            aggsh.at[pl.ds(NS * ST, TAIL)], out.at[pl.ds(c * N + NS * ST, TAIL)]
        )


BN = 1000


def _norm(deg_block):
    d = deg_block[:, 0:1]
    return jnp.where(d > 0.0, lax.rsqrt(jnp.maximum(d, 1.0)), 0.0)


def _mm1_body(x_ref, w_ref, dsrc_ref, o_ref):
    ns = _norm(dsrc_ref)
    y = jnp.dot(
        x_ref[...], w_ref[...],
        preferred_element_type=jnp.float32, precision=lax.Precision.HIGHEST,
    )
    o_ref[0] = y * ns


def _mm1(x, w1, dsrc):
    return pl.pallas_call(
        _mm1_body,
        grid=(2, N // BN),
        in_specs=[
            pl.BlockSpec((BN, DI), lambda c, i: (i, 0)),
            pl.BlockSpec((DI, HALF), lambda c, i: (0, c)),
            pl.BlockSpec((BN, 16), lambda c, i: (i, 0)),
        ],
        out_specs=pl.BlockSpec((1, BN, HALF), lambda c, i: (c, i, 0)),
        out_shape=jax.ShapeDtypeStruct((2, N, HALF), jnp.float32),
    )(x, w1, dsrc)


def _mm2_body(agg_ref, dsrc_ref, ddst_ref, b1_ref, w2_ref, o_ref):
    nd = _norm(ddst_ref)
    ns = _norm(dsrc_ref)
    a = jnp.concatenate([agg_ref[0], agg_ref[1]], axis=1)
    z = jnp.maximum(a * nd + b1_ref[...], 0.0)
    y = jnp.dot(
        z, w2_ref[...],
        preferred_element_type=jnp.float32, precision=lax.Precision.HIGHEST,
    )
    o_ref[0] = y * ns


def _mm2(agg1, dsrc, ddst, b1, w2):
    return pl.pallas_call(
        _mm2_body,
        grid=(2, N // BN),
        in_specs=[
            pl.BlockSpec((2, BN, HALF), lambda c, i: (0, i, 0)),
            pl.BlockSpec((BN, 16), lambda c, i: (i, 0)),
            pl.BlockSpec((BN, 16), lambda c, i: (i, 0)),
            pl.BlockSpec((1, DH), lambda c, i: (0, 0)),
            pl.BlockSpec((DH, HALF), lambda c, i: (0, c)),
        ],
        out_specs=pl.BlockSpec((1, BN, HALF), lambda c, i: (c, i, 0)),
        out_shape=jax.ShapeDtypeStruct((2, N, HALF), jnp.float32),
    )(agg1, dsrc, ddst, b1, w2)


def _mm3_body(agg_ref, ddst_ref, b2_ref, o_ref):
    nd = _norm(ddst_ref)
    a = jnp.concatenate([agg_ref[0], agg_ref[1]], axis=1)
    o_ref[...] = a * nd + b2_ref[...]


def _mm3(agg2, ddst, b2):
    return pl.pallas_call(
        _mm3_body,
        grid=(N // BN,),
        in_specs=[
            pl.BlockSpec((2, BN, HALF), lambda i: (0, i, 0)),
            pl.BlockSpec((BN, 16), lambda i: (i, 0)),
            pl.BlockSpec((1, DH), lambda i: (0, 0)),
        ],
        out_specs=pl.BlockSpec((BN, DH), lambda i: (i, 0)),
        out_shape=jax.ShapeDtypeStruct((N, DH), jnp.float32),
    )(agg2, ddst, b2)


def kernel(features, edge_index, W1, b1, W2, b2):
    src = edge_index[0]
    dst = edge_index[1]
    edges = jnp.concatenate([src, src + N, dst])

    zdeg = jnp.zeros((N, 16), jnp.float32)
    zagg = jnp.zeros((N, HALF), jnp.float32)

    degs = _deg_kernel(edges, zdeg)
    dsrc = degs[:N]
    ddst = degs[N:]

    h1 = _mm1(features, W1, dsrc)
    agg1 = _agg_kernel(h1.reshape(2 * N, HALF), edges, zagg).reshape(2, N, HALF)

    h2 = _mm2(agg1, dsrc, ddst, b1.reshape(1, DH), W2)
    agg2 = _agg_kernel(h2.reshape(2 * N, HALF), edges, zagg).reshape(2, N, HALF)

    return _mm3(agg2, ddst, b2.reshape(1, DH))


# speedup vs baseline: 7.4003x; 1.3485x over previous
import functools

import jax
import jax.numpy as jnp
from jax import lax
from jax.experimental import pallas as pl
from jax.experimental.pallas import tpu as pltpu
from jax.experimental.pallas import tpu_sc as plsc

N = 10000
E = 320000
DI = 128
DH = 256
HALF = DH // 2

NC = 2
NS = 16
EPT = E // NS
CH = 160
NCHUNK = EPT // CH
NPAIR = NCHUNK // 2
ST = 624
TAIL = N - NS * ST

_MESH = plsc.VectorSubcoreMesh(
    core_axis_name="c", subcore_axis_name="s", num_cores=NC, num_subcores=NS
)


@functools.partial(
    pl.kernel,
    out_type=jax.ShapeDtypeStruct((2 * N, 16), jnp.float32),
    mesh=_MESH,
    scratch_types=[
        pltpu.VMEM((CH,), jnp.int32),
        pltpu.VMEM((CH, 16), jnp.float32),
        pltpu.VMEM_SHARED((N, 16), jnp.float32),
    ],
    compiler_params=pltpu.CompilerParams(use_tc_tiling_on_sc=False),
)
def _deg_kernel(edges, zdeg, out, idxv, onesv, degsh):
    c = lax.axis_index("c")
    s = lax.axis_index("s")

    @pl.loop(0, CH)
    def _(i):
        onesv[i] = jnp.full((16,), 1.0, jnp.float32)

    pltpu.sync_copy(zdeg.at[pl.ds(s * ST, ST)], degsh.at[pl.ds(s * ST, ST)])

    @pl.when(s == NS - 1)
    def _():
        pltpu.sync_copy(
            zdeg.at[pl.ds(NS * ST, TAIL)], degsh.at[pl.ds(NS * ST, TAIL)]
        )

    plsc.subcore_barrier()

    base = c * (2 * E) + s * EPT

    @pl.loop(0, NCHUNK)
    def _(i):
        pltpu.sync_copy(edges.at[pl.ds(base + i * CH, CH)], idxv)
        pltpu.sync_copy(onesv, degsh.at[idxv], add=True)

    plsc.subcore_barrier()
    pltpu.sync_copy(
        degsh.at[pl.ds(s * ST, ST)], out.at[pl.ds(c * N + s * ST, ST)]
    )

    @pl.when(s == NS - 1)
    def _():
        pltpu.sync_copy(
            degsh.at[pl.ds(NS * ST, TAIL)], out.at[pl.ds(c * N + NS * ST, TAIL)]
        )


@functools.partial(
    pl.kernel,
    out_type=jax.ShapeDtypeStruct((2 * N, HALF), jnp.float32),
    mesh=_MESH,
    scratch_types=[
        pltpu.VMEM((CH,), jnp.int32),
        pltpu.VMEM((CH,), jnp.int32),
        pltpu.VMEM((CH,), jnp.int32),
        pltpu.VMEM((CH,), jnp.int32),
        pltpu.VMEM((CH, HALF), jnp.float32),
        pltpu.VMEM((CH, HALF), jnp.float32),
        pltpu.SemaphoreType.DMA,
        pltpu.SemaphoreType.DMA,
        pltpu.VMEM_SHARED((N, HALF), jnp.float32),
    ],
)
def _agg_kernel(h, edges, zagg, out, isrc0, idst0, isrc1, idst1, rows0,
                rows1, gs0, gs1, aggsh):
    c = lax.axis_index("c")
    s = lax.axis_index("s")

    pltpu.sync_copy(zagg.at[pl.ds(s * ST, ST)], aggsh.at[pl.ds(s * ST, ST)])

    @pl.when(s == NS - 1)
    def _():
        pltpu.sync_copy(
            zagg.at[pl.ds(NS * ST, TAIL)], aggsh.at[pl.ds(NS * ST, TAIL)]
        )

    plsc.subcore_barrier()

    bs = c * E + s * EPT
    bd = 2 * E + s * EPT

    def load_idx(i, isrc, idst):
        pltpu.sync_copy(edges.at[pl.ds(bs + i * CH, CH)], isrc)
        pltpu.sync_copy(edges.at[pl.ds(bd + i * CH, CH)], idst)

    def start_gather(isrc, rows, sem):
        pltpu.make_async_copy(h.at[isrc], rows, sem).start()

    def wait_gather(isrc, rows, sem):
        pltpu.make_async_copy(h.at[isrc], rows, sem).wait()

    load_idx(0, isrc0, idst0)
    start_gather(isrc0, rows0, gs0)
    load_idx(1, isrc1, idst1)

    @pl.loop(0, NPAIR)
    def _(t):
        start_gather(isrc1, rows1, gs1)
        wait_gather(isrc0, rows0, gs0)
        pltpu.sync_copy(rows0, aggsh.at[idst0], add=True)
        load_idx(2 * t + 2, isrc0, idst0)
        start_gather(isrc0, rows0, gs0)
        wait_gather(isrc1, rows1, gs1)
        pltpu.sync_copy(rows1, aggsh.at[idst1], add=True)

        @pl.when(2 * t + 3 < NCHUNK)
        def _():
            load_idx(2 * t + 3, isrc1, idst1)

    wait_gather(isrc0, rows0, gs0)
    pltpu.sync_copy(rows0, aggsh.at[idst0], add=True)

    plsc.subcore_barrier()
    pltpu.sync_copy(
        aggsh.at[pl.ds(s * ST, ST)], out.at[pl.ds(c * N + s * ST, ST)]
    )

    @pl.when(s == NS - 1)
    def _():
        pltpu.sync_copy(
            aggsh.at[pl.ds(NS * ST, TAIL)], out.at[pl.ds(c * N + NS * ST, TAIL)]
        )


BN = 1000


def _norm(deg_block):
    d = deg_block[:, 0:1]
    return jnp.where(d > 0.0, lax.rsqrt(jnp.maximum(d, 1.0)), 0.0)


def _mm1_body(x_ref, w_ref, dsrc_ref, o_ref):
    ns = _norm(dsrc_ref)
    y = jnp.dot(
        x_ref[...], w_ref[...],
        preferred_element_type=jnp.float32, precision=lax.Precision.HIGHEST,
    )
    o_ref[0] = y * ns


def _mm1(x, w1, dsrc):
    return pl.pallas_call(
        _mm1_body,
        grid=(2, N // BN),
        in_specs=[
            pl.BlockSpec((BN, DI), lambda c, i: (i, 0)),
            pl.BlockSpec((DI, HALF), lambda c, i: (0, c)),
            pl.BlockSpec((BN, 16), lambda c, i: (i, 0)),
        ],
        out_specs=pl.BlockSpec((1, BN, HALF), lambda c, i: (c, i, 0)),
        out_shape=jax.ShapeDtypeStruct((2, N, HALF), jnp.float32),
    )(x, w1, dsrc)


def _mm2_body(agg_ref, dsrc_ref, ddst_ref, b1_ref, w2_ref, o_ref):
    nd = _norm(ddst_ref)
    ns = _norm(dsrc_ref)
    a = jnp.concatenate([agg_ref[0], agg_ref[1]], axis=1)
    z = jnp.maximum(a * nd + b1_ref[...], 0.0)
    y = jnp.dot(
        z, w2_ref[...],
        preferred_element_type=jnp.float32, precision=lax.Precision.HIGHEST,
    )
    o_ref[0] = y * ns


def _mm2(agg1, dsrc, ddst, b1, w2):
    return pl.pallas_call(
        _mm2_body,
        grid=(2, N // BN),
        in_specs=[
            pl.BlockSpec((2, BN, HALF), lambda c, i: (0, i, 0)),
            pl.BlockSpec((BN, 16), lambda c, i: (i, 0)),
            pl.BlockSpec((BN, 16), lambda c, i: (i, 0)),
            pl.BlockSpec((1, DH), lambda c, i: (0, 0)),
            pl.BlockSpec((DH, HALF), lambda c, i: (0, c)),
        ],
        out_specs=pl.BlockSpec((1, BN, HALF), lambda c, i: (c, i, 0)),
        out_shape=jax.ShapeDtypeStruct((2, N, HALF), jnp.float32),
    )(agg1, dsrc, ddst, b1, w2)


def _mm3_body(agg_ref, ddst_ref, b2_ref, o_ref):
    nd = _norm(ddst_ref)
    a = jnp.concatenate([agg_ref[0], agg_ref[1]], axis=1)
    o_ref[...] = a * nd + b2_ref[...]


def _mm3(agg2, ddst, b2):
    return pl.pallas_call(
        _mm3_body,
        grid=(N // BN,),
        in_specs=[
            pl.BlockSpec((2, BN, HALF), lambda i: (0, i, 0)),
            pl.BlockSpec((BN, 16), lambda i: (i, 0)),
            pl.BlockSpec((1, DH), lambda i: (0, 0)),
        ],
        out_specs=pl.BlockSpec((BN, DH), lambda i: (i, 0)),
        out_shape=jax.ShapeDtypeStruct((N, DH), jnp.float32),
    )(agg2, ddst, b2)


def kernel(features, edge_index, W1, b1, W2, b2):
    src = edge_index[0]
    dst = edge_index[1]
    edges = jnp.concatenate([src, src + N, dst])

    zdeg = jnp.zeros((N, 16), jnp.float32)
    zagg = jnp.zeros((N, HALF), jnp.float32)

    degs = _deg_kernel(edges, zdeg)
    dsrc = degs[:N]
    ddst = degs[N:]

    h1 = _mm1(features, W1, dsrc)
    agg1 = _agg_kernel(h1.reshape(2 * N, HALF), edges, zagg).reshape(2, N, HALF)

    h2 = _mm2(agg1, dsrc, ddst, b1.reshape(1, DH), W2)
    agg2 = _agg_kernel(h2.reshape(2 * N, HALF), edges, zagg).reshape(2, N, HALF)

    return _mm3(agg2, ddst, b2.reshape(1, DH))


# speedup vs baseline: 8.9935x; 1.2153x over previous
import functools

import jax
import jax.numpy as jnp
from jax import lax
from jax.experimental import pallas as pl
from jax.experimental.pallas import tpu as pltpu
from jax.experimental.pallas import tpu_sc as plsc

N = 10000
E = 320000
DI = 128
DH = 256
HALF = DH // 2

NC = 2
NS = 16
EPT = E // NS
CH = 160
NCHUNK = EPT // CH
NPAIR = NCHUNK // 2
CHD = 2000
NCHUNKD = EPT // CHD
ST = 624
TAIL = N - NS * ST

_MESH = plsc.VectorSubcoreMesh(
    core_axis_name="c", subcore_axis_name="s", num_cores=NC, num_subcores=NS
)


@functools.partial(
    pl.kernel,
    out_type=jax.ShapeDtypeStruct((2 * N, 16), jnp.float32),
    mesh=_MESH,
    scratch_types=[
        pltpu.VMEM((CHD,), jnp.int32),
        pltpu.VMEM((CHD, 16), jnp.float32),
        pltpu.VMEM_SHARED((N, 16), jnp.float32),
    ],
    compiler_params=pltpu.CompilerParams(use_tc_tiling_on_sc=False),
)
def _deg_kernel(edges, zdeg, out, idxv, onesv, degsh):
    c = lax.axis_index("c")
    s = lax.axis_index("s")

    @pl.loop(0, CHD)
    def _(i):
        onesv[i] = jnp.full((16,), 1.0, jnp.float32)

    pltpu.sync_copy(zdeg.at[pl.ds(s * ST, ST)], degsh.at[pl.ds(s * ST, ST)])

    @pl.when(s == NS - 1)
    def _():
        pltpu.sync_copy(
            zdeg.at[pl.ds(NS * ST, TAIL)], degsh.at[pl.ds(NS * ST, TAIL)]
        )

    plsc.subcore_barrier()

    base = c * (2 * E) + s * EPT

    @pl.loop(0, NCHUNKD)
    def _(i):
        pltpu.sync_copy(edges.at[pl.ds(base + i * CHD, CHD)], idxv)
        pltpu.sync_copy(onesv, degsh.at[idxv], add=True)

    plsc.subcore_barrier()
    pltpu.sync_copy(
        degsh.at[pl.ds(s * ST, ST)], out.at[pl.ds(c * N + s * ST, ST)]
    )

    @pl.when(s == NS - 1)
    def _():
        pltpu.sync_copy(
            degsh.at[pl.ds(NS * ST, TAIL)], out.at[pl.ds(c * N + NS * ST, TAIL)]
        )


@functools.partial(
    pl.kernel,
    out_type=jax.ShapeDtypeStruct((2 * N, HALF), jnp.float32),
    mesh=_MESH,
    scratch_types=[
        pltpu.VMEM((2 * CH,), jnp.int32),
        pltpu.VMEM((2 * CH,), jnp.int32),
        pltpu.VMEM((CH, HALF), jnp.float32),
        pltpu.VMEM((CH, HALF), jnp.float32),
        pltpu.SemaphoreType.DMA,
        pltpu.SemaphoreType.DMA,
        pltpu.VMEM_SHARED((N, HALF), jnp.float32),
    ],
    compiler_params=pltpu.CompilerParams(use_tc_tiling_on_sc=False),
)
def _agg_kernel(h, edges, zagg, out, ibuf0, ibuf1, rows0,
                rows1, gs0, gs1, aggsh):
    c = lax.axis_index("c")
    s = lax.axis_index("s")

    pltpu.sync_copy(zagg.at[pl.ds(s * ST, ST)], aggsh.at[pl.ds(s * ST, ST)])

    @pl.when(s == NS - 1)
    def _():
        pltpu.sync_copy(
            zagg.at[pl.ds(NS * ST, TAIL)], aggsh.at[pl.ds(NS * ST, TAIL)]
        )

    plsc.subcore_barrier()

    base = (c * NS + s) * EPT * 2

    def load_idx(i, ibuf):
        pltpu.sync_copy(edges.at[pl.ds(base + i * 2 * CH, 2 * CH)], ibuf)

    def start_gather(ibuf, rows, sem):
        pltpu.make_async_copy(h.at[ibuf.at[pl.ds(0, CH)]], rows, sem).start()

    def wait_gather(ibuf, rows, sem):
        pltpu.make_async_copy(h.at[ibuf.at[pl.ds(0, CH)]], rows, sem).wait()

    def scatter_add(rows, ibuf):
        pltpu.sync_copy(rows, aggsh.at[ibuf.at[pl.ds(CH, CH)]], add=True)

    load_idx(0, ibuf0)
    start_gather(ibuf0, rows0, gs0)
    load_idx(1, ibuf1)

    @pl.loop(0, NPAIR)
    def _(t):
        start_gather(ibuf1, rows1, gs1)
        wait_gather(ibuf0, rows0, gs0)
        scatter_add(rows0, ibuf0)
        load_idx(2 * t + 2, ibuf0)
        start_gather(ibuf0, rows0, gs0)
        wait_gather(ibuf1, rows1, gs1)
        scatter_add(rows1, ibuf1)

        @pl.when(2 * t + 3 < NCHUNK)
        def _():
            load_idx(2 * t + 3, ibuf1)

    wait_gather(ibuf0, rows0, gs0)
    scatter_add(rows0, ibuf0)

    plsc.subcore_barrier()
    pltpu.sync_copy(
        aggsh.at[pl.ds(s * ST, ST)], out.at[pl.ds(c * N + s * ST, ST)]
    )

    @pl.when(s == NS - 1)
    def _():
        pltpu.sync_copy(
            aggsh.at[pl.ds(NS * ST, TAIL)], out.at[pl.ds(c * N + NS * ST, TAIL)]
        )


BN = 1000


def _norm(deg_block):
    d = deg_block[:, 0:1]
    return jnp.where(d > 0.0, lax.rsqrt(jnp.maximum(d, 1.0)), 0.0)


def _mm1_body(x_ref, w_ref, dsrc_ref, o_ref):
    ns = _norm(dsrc_ref)
    y = jnp.dot(
        x_ref[...], w_ref[...],
        preferred_element_type=jnp.float32, precision=lax.Precision.HIGHEST,
    )
    o_ref[0] = y * ns


def _mm1(x, w1, dsrc):
    return pl.pallas_call(
        _mm1_body,
        grid=(2, N // BN),
        in_specs=[
            pl.BlockSpec((BN, DI), lambda c, i: (i, 0)),
            pl.BlockSpec((DI, HALF), lambda c, i: (0, c)),
            pl.BlockSpec((BN, 16), lambda c, i: (i, 0)),
        ],
        out_specs=pl.BlockSpec((1, BN, HALF), lambda c, i: (c, i, 0)),
        out_shape=jax.ShapeDtypeStruct((2, N, HALF), jnp.float32),
    )(x, w1, dsrc)


def _mm2_body(agg_ref, dsrc_ref, ddst_ref, b1_ref, w2_ref, o_ref):
    nd = _norm(ddst_ref)
    ns = _norm(dsrc_ref)
    a = jnp.concatenate([agg_ref[0], agg_ref[1]], axis=1)
    z = jnp.maximum(a * nd + b1_ref[...], 0.0)
    y = jnp.dot(
        z, w2_ref[...],
        preferred_element_type=jnp.float32, precision=lax.Precision.HIGHEST,
    )
    o_ref[0] = y * ns


def _mm2(agg1, dsrc, ddst, b1, w2):
    return pl.pallas_call(
        _mm2_body,
        grid=(2, N // BN),
        in_specs=[
            pl.BlockSpec((2, BN, HALF), lambda c, i: (0, i, 0)),
            pl.BlockSpec((BN, 16), lambda c, i: (i, 0)),
            pl.BlockSpec((BN, 16), lambda c, i: (i, 0)),
            pl.BlockSpec((1, DH), lambda c, i: (0, 0)),
            pl.BlockSpec((DH, HALF), lambda c, i: (0, c)),
        ],
        out_specs=pl.BlockSpec((1, BN, HALF), lambda c, i: (c, i, 0)),
        out_shape=jax.ShapeDtypeStruct((2, N, HALF), jnp.float32),
    )(agg1, dsrc, ddst, b1, w2)


def _mm3_body(agg_ref, ddst_ref, b2_ref, o_ref):
    nd = _norm(ddst_ref)
    a = jnp.concatenate([agg_ref[0], agg_ref[1]], axis=1)
    o_ref[...] = a * nd + b2_ref[...]


def _mm3(agg2, ddst, b2):
    return pl.pallas_call(
        _mm3_body,
        grid=(N // BN,),
        in_specs=[
            pl.BlockSpec((2, BN, HALF), lambda i: (0, i, 0)),
            pl.BlockSpec((BN, 16), lambda i: (i, 0)),
            pl.BlockSpec((1, DH), lambda i: (0, 0)),
        ],
        out_specs=pl.BlockSpec((BN, DH), lambda i: (i, 0)),
        out_shape=jax.ShapeDtypeStruct((N, DH), jnp.float32),
    )(agg2, ddst, b2)


def kernel(features, edge_index, W1, b1, W2, b2):
    src = edge_index[0]
    dst = edge_index[1]
    edges = jnp.concatenate([src, src + N, dst])

    srcw = src.reshape(NS, NCHUNK, 1, CH)
    dstw = dst.reshape(NS, NCHUNK, 1, CH)
    core0 = jnp.concatenate([srcw, dstw], axis=2)
    core1 = jnp.concatenate([srcw + N, dstw], axis=2)
    edges_ag = jnp.stack([core0, core1]).reshape(-1)

    zdeg = jnp.zeros((N, 16), jnp.float32)
    zagg = jnp.zeros((N, HALF), jnp.float32)

    degs = _deg_kernel(edges, zdeg)
    dsrc = degs[:N]
    ddst = degs[N:]

    h1 = _mm1(features, W1, dsrc)
    agg1 = _agg_kernel(
        h1.reshape(2 * N, HALF), edges_ag, zagg).reshape(2, N, HALF)

    h2 = _mm2(agg1, dsrc, ddst, b1.reshape(1, DH), W2)
    agg2 = _agg_kernel(
        h2.reshape(2 * N, HALF), edges_ag, zagg).reshape(2, N, HALF)

    return _mm3(agg2, ddst, b2.reshape(1, DH))


# speedup vs baseline: 9.0022x; 1.0010x over previous
import functools

import numpy as np
import jax
import jax.numpy as jnp
from jax import lax
from jax.experimental import pallas as pl
from jax.experimental.pallas import tpu as pltpu
from jax.experimental.pallas import tpu_sc as plsc

N = 10000
E = 320000
DI = 128
DH = 256
HALF = DH // 2

NC = 2
NS = 16
EPT = E // NS
CH = 160
NCHUNK = EPT // CH
NPAIR = NCHUNK // 2
CHD = 2000
NCHUNKD = EPT // CHD
ST = 624
TAIL = N - NS * ST

_MESH = plsc.VectorSubcoreMesh(
    core_axis_name="c", subcore_axis_name="s", num_cores=NC, num_subcores=NS
)


@functools.partial(
    pl.kernel,
    out_type=jax.ShapeDtypeStruct((2 * N, 16), jnp.float32),
    mesh=_MESH,
    scratch_types=[
        pltpu.VMEM((CHD,), jnp.int32),
        pltpu.VMEM((CHD, 16), jnp.float32),
        pltpu.VMEM_SHARED((N, 16), jnp.float32),
    ],
    compiler_params=pltpu.CompilerParams(use_tc_tiling_on_sc=False),
)
def _deg_kernel(edges, zdeg, out, idxv, onesv, degsh):
    c = lax.axis_index("c")
    s = lax.axis_index("s")

    @pl.loop(0, CHD)
    def _(i):
        onesv[i] = jnp.full((16,), 1.0, jnp.float32)

    pltpu.sync_copy(zdeg.at[pl.ds(s * ST, ST)], degsh.at[pl.ds(s * ST, ST)])

    @pl.when(s == NS - 1)
    def _():
        pltpu.sync_copy(
            zdeg.at[pl.ds(NS * ST, TAIL)], degsh.at[pl.ds(NS * ST, TAIL)]
        )

    plsc.subcore_barrier()

    base = c * E + s * EPT

    @pl.loop(0, NCHUNKD)
    def _(i):
        pltpu.sync_copy(edges.at[pl.ds(base + i * CHD, CHD)], idxv)
        pltpu.sync_copy(onesv, degsh.at[idxv], add=True)

    plsc.subcore_barrier()
    pltpu.sync_copy(
        degsh.at[pl.ds(s * ST, ST)], out.at[pl.ds(c * N + s * ST, ST)]
    )

    @pl.when(s == NS - 1)
    def _():
        pltpu.sync_copy(
            degsh.at[pl.ds(NS * ST, TAIL)], out.at[pl.ds(c * N + NS * ST, TAIL)]
        )


@functools.partial(
    pl.kernel,
    out_type=jax.ShapeDtypeStruct((2 * N, HALF), jnp.float32),
    mesh=_MESH,
    scratch_types=[
        pltpu.VMEM((2 * CH,), jnp.int32),
        pltpu.VMEM((2 * CH,), jnp.int32),
        pltpu.VMEM((CH, HALF), jnp.float32),
        pltpu.VMEM((CH, HALF), jnp.float32),
        pltpu.SemaphoreType.DMA,
        pltpu.SemaphoreType.DMA,
        pltpu.VMEM_SHARED((N, HALF), jnp.float32),
    ],
    compiler_params=pltpu.CompilerParams(use_tc_tiling_on_sc=False),
)
def _agg_kernel(h, edges, zagg, out, ibuf0, ibuf1, rows0,
                rows1, gs0, gs1, aggsh):
    c = lax.axis_index("c")
    s = lax.axis_index("s")

    pltpu.sync_copy(zagg.at[pl.ds(s * ST, ST)], aggsh.at[pl.ds(s * ST, ST)])

    @pl.when(s == NS - 1)
    def _():
        pltpu.sync_copy(
            zagg.at[pl.ds(NS * ST, TAIL)], aggsh.at[pl.ds(NS * ST, TAIL)]
        )

    plsc.subcore_barrier()

    base = (c * NS + s) * EPT * 2

    def load_idx(i, ibuf):
        pltpu.sync_copy(edges.at[pl.ds(base + i * 2 * CH, 2 * CH)], ibuf)

    def start_gather(ibuf, rows, sem):
        pltpu.make_async_copy(h.at[ibuf.at[pl.ds(0, CH)]], rows, sem).start()

    def wait_gather(ibuf, rows, sem):
        pltpu.make_async_copy(h.at[ibuf.at[pl.ds(0, CH)]], rows, sem).wait()

    def scatter_add(rows, ibuf):
        pltpu.sync_copy(rows, aggsh.at[ibuf.at[pl.ds(CH, CH)]], add=True)

    load_idx(0, ibuf0)
    start_gather(ibuf0, rows0, gs0)
    load_idx(1, ibuf1)

    @pl.loop(0, NPAIR)
    def _(t):
        start_gather(ibuf1, rows1, gs1)
        wait_gather(ibuf0, rows0, gs0)
        scatter_add(rows0, ibuf0)
        load_idx(2 * t + 2, ibuf0)
        start_gather(ibuf0, rows0, gs0)
        wait_gather(ibuf1, rows1, gs1)
        scatter_add(rows1, ibuf1)

        @pl.when(2 * t + 3 < NCHUNK)
        def _():
            load_idx(2 * t + 3, ibuf1)

    wait_gather(ibuf0, rows0, gs0)
    scatter_add(rows0, ibuf0)

    plsc.subcore_barrier()
    pltpu.sync_copy(
        aggsh.at[pl.ds(s * ST, ST)], out.at[pl.ds(c * N + s * ST, ST)]
    )

    @pl.when(s == NS - 1)
    def _():
        pltpu.sync_copy(
            aggsh.at[pl.ds(NS * ST, TAIL)], out.at[pl.ds(c * N + NS * ST, TAIL)]
        )


BN = 1000


def _norm(deg_block):
    d = deg_block[:, 0:1]
    return jnp.where(d > 0.0, lax.rsqrt(jnp.maximum(d, 1.0)), 0.0)


def _mm1_body(x_ref, w_ref, dsrc_ref, o_ref):
    ns = _norm(dsrc_ref)
    y = jnp.dot(
        x_ref[...], w_ref[...],
        preferred_element_type=jnp.float32, precision=lax.Precision.HIGHEST,
    )
    o_ref[0] = y * ns


def _mm1(x, w1, dsrc):
    return pl.pallas_call(
        _mm1_body,
        grid=(2, N // BN),
        in_specs=[
            pl.BlockSpec((BN, DI), lambda c, i: (i, 0)),
            pl.BlockSpec((DI, HALF), lambda c, i: (0, c)),
            pl.BlockSpec((BN, 16), lambda c, i: (i, 0)),
        ],
        out_specs=pl.BlockSpec((1, BN, HALF), lambda c, i: (c, i, 0)),
        out_shape=jax.ShapeDtypeStruct((2, N, HALF), jnp.float32),
    )(x, w1, dsrc)


def _mm2_body(agg_ref, dsrc_ref, ddst_ref, b1_ref, w2_ref, o_ref):
    nd = _norm(ddst_ref)
    ns = _norm(dsrc_ref)
    a = jnp.concatenate([agg_ref[0], agg_ref[1]], axis=1)
    z = jnp.maximum(a * nd + b1_ref[...], 0.0)
    y = jnp.dot(
        z, w2_ref[...],
        preferred_element_type=jnp.float32, precision=lax.Precision.HIGHEST,
    )
    o_ref[0] = y * ns


def _mm2(agg1, dsrc, ddst, b1, w2):
    return pl.pallas_call(
        _mm2_body,
        grid=(2, N // BN),
        in_specs=[
            pl.BlockSpec((2, BN, HALF), lambda c, i: (0, i, 0)),
            pl.BlockSpec((BN, 16), lambda c, i: (i, 0)),
            pl.BlockSpec((BN, 16), lambda c, i: (i, 0)),
            pl.BlockSpec((1, DH), lambda c, i: (0, 0)),
            pl.BlockSpec((DH, HALF), lambda c, i: (0, c)),
        ],
        out_specs=pl.BlockSpec((1, BN, HALF), lambda c, i: (c, i, 0)),
        out_shape=jax.ShapeDtypeStruct((2, N, HALF), jnp.float32),
    )(agg1, dsrc, ddst, b1, w2)


def _mm3_body(agg_ref, ddst_ref, b2_ref, o_ref):
    nd = _norm(ddst_ref)
    a = jnp.concatenate([agg_ref[0], agg_ref[1]], axis=1)
    o_ref[...] = a * nd + b2_ref[...]


def _mm3(agg2, ddst, b2):
    return pl.pallas_call(
        _mm3_body,
        grid=(N // BN,),
        in_specs=[
            pl.BlockSpec((2, BN, HALF), lambda i: (0, i, 0)),
            pl.BlockSpec((BN, 16), lambda i: (i, 0)),
            pl.BlockSpec((1, DH), lambda i: (0, 0)),
        ],
        out_specs=pl.BlockSpec((BN, DH), lambda i: (i, 0)),
        out_shape=jax.ShapeDtypeStruct((N, DH), jnp.float32),
    )(agg2, ddst, b2)


_ZDEG = np.zeros((N, 16), np.float32)
_ZAGG = np.zeros((N, HALF), np.float32)


def kernel(features, edge_index, W1, b1, W2, b2):
    src = edge_index[0]
    dst = edge_index[1]
    edges = edge_index.reshape(-1)

    srcw = src.reshape(NS, NCHUNK, 1, CH)
    dstw = dst.reshape(NS, NCHUNK, 1, CH)
    core0 = jnp.concatenate([srcw, dstw], axis=2)
    core1 = jnp.concatenate([srcw + N, dstw], axis=2)
    edges_ag = jnp.stack([core0, core1]).reshape(-1)

    zdeg = jnp.asarray(_ZDEG)
    zagg = jnp.asarray(_ZAGG)

    degs = _deg_kernel(edges, zdeg)
    dsrc = degs[:N]
    ddst = degs[N:]

    h1 = _mm1(features, W1, dsrc)
    agg1 = _agg_kernel(
        h1.reshape(2 * N, HALF), edges_ag, zagg).reshape(2, N, HALF)

    h2 = _mm2(agg1, dsrc, ddst, b1.reshape(1, DH), W2)
    agg2 = _agg_kernel(
        h2.reshape(2 * N, HALF), edges_ag, zagg).reshape(2, N, HALF)

    return _mm3(agg2, ddst, b2.reshape(1, DH))
